# Initial kernel scaffold; baseline (speedup 1.0000x reference)
import jax
import jax.numpy as jnp
from jax.experimental import pallas as pl


def kernel(x, cantor_positions, W_qkv, b_qkv, W_out, b_out):
    raise NotImplementedError("write your pallas kernel here")


# speedup vs baseline: 15.3559x; 15.3559x over previous
import functools
import math

import jax
import jax.numpy as jnp
from jax import lax
from jax.experimental import pallas as pl
from jax.experimental.pallas import tpu as pltpu
from jax.experimental.pallas import tpu_sc as plsc

S = 2048
DIM = 1024
HEADS, HD = 16, 64
KN = 64
SCALE = 1.0 / math.sqrt(HD)
QBLK = 128
WIN = 272
RBLK = 256
BIG = 1.0e30

_MM_PREC = jax.lax.Precision.HIGHEST


def _routing_body(cp_col_ref, cp_row_ref, order_ref, rank_ref, lo_ref,
                  ex_ref, ad_ref, s_acc, pad_ref, ord_pad_ref):
    g = pl.program_id(0)

    @pl.when(g == 0)
    def _init():
        s_acc[...] = jnp.zeros_like(s_acc)
        order_ref[...] = jnp.zeros_like(order_ref)

    ci = cp_col_ref[...]
    cr = cp_row_ref[...]
    jidx = lax.broadcasted_iota(jnp.int32, (1, S), 1)
    iglob = g * RBLK + lax.broadcasted_iota(jnp.int32, (RBLK, 1), 0)

    lt = (cr < ci).astype(jnp.int32)
    eq = jnp.logical_and(cr == ci, jidx < iglob).astype(jnp.int32)
    rank = jnp.sum(lt + eq, axis=1, keepdims=True)
    rank_ref[...] = rank

    onehot = jidx == rank
    s_acc[...] += jnp.sum(jnp.where(onehot, ci, 0.0), axis=0, keepdims=True)
    order_ref[...] += jnp.sum(jnp.where(onehot, iglob, 0), axis=0,
                              keepdims=True)

    @pl.when(g == pl.num_programs(0) - 1)
    def _window_starts():
        s = s_acc[...]
        pad_ref[:, 0:KN] = jnp.full((1, KN), -BIG, jnp.float32)
        pad_ref[:, KN:KN + S] = s
        pad_ref[:, KN + S:] = jnp.full((1, KN), BIG, jnp.float32)
        ord_pad_ref[:, 0:KN] = jnp.full((1, KN), 1e9, jnp.float32)
        ord_pad_ref[:, KN:KN + S] = order_ref[...].astype(jnp.float32)
        ord_pad_ref[:, KN + S:] = jnp.full((1, KN), 1e9, jnp.float32)
        best = jnp.full((1, S), 2.0 * BIG, jnp.float32)
        bt = jnp.zeros((1, S), jnp.int32)
        b_sl = jnp.zeros((1, S), jnp.float32)
        b_sll = jnp.zeros((1, S), jnp.float32)
        b_sr = jnp.zeros((1, S), jnp.float32)
        b_sr2 = jnp.zeros((1, S), jnp.float32)
        b_ol = jnp.zeros((1, S), jnp.float32)
        b_or = jnp.zeros((1, S), jnp.float32)
        for t in range(KN):
            a = pad_ref[:, KN - t:KN - t + S]
            al = pad_ref[:, KN - t - 1:KN - t - 1 + S]
            b = pad_ref[:, KN + KN - 1 - t:KN + KN - 1 - t + S]
            b2 = pad_ref[:, KN + KN - 2 - t:KN + KN - 2 - t + S]
            cost = jnp.maximum(s - a, b - s)
            take = cost < best
            best = jnp.where(take, cost, best)
            bt = jnp.where(take, t, bt)
            b_sl = jnp.where(take, a, b_sl)
            b_sll = jnp.where(take, al, b_sll)
            b_sr = jnp.where(take, b, b_sr)
            b_sr2 = jnp.where(take, b2, b_sr2)
            b_ol = jnp.where(
                take, ord_pad_ref[:, KN - t - 1:KN - t - 1 + S], b_ol)
            b_or = jnp.where(
                take, ord_pad_ref[:, KN + KN - 1 - t:KN + KN - 1 - t + S],
                b_or)
        iidx = lax.broadcasted_iota(jnp.int32, (1, S), 1)
        lo = jnp.clip(iidx - bt, 0, S - KN)
        b_dl = s - b_sll
        b_dr = b_sr - s
        case2 = jnp.logical_and(b_sll == b_sl, (s - b_sl) == best)
        case1 = jnp.logical_and(
            jnp.logical_not(case2),
            jnp.logical_and(b_dl == b_dr, b_ol < b_or))
        case3 = jnp.logical_and(
            jnp.logical_and(b_sr2 == b_sr, b_dr == best),
            jnp.logical_and(b_dl < b_dr,
                            jnp.logical_not(jnp.logical_or(case1, case2))))
        sw_r = jnp.logical_or(case1, case3)
        ex = jnp.where(case2, lo, jnp.where(sw_r, lo + KN - 1, -1))
        ad = jnp.where(jnp.logical_or(sw_r, case2), lo - 1, -1)
        lo_ref[...] = lo
        ex_ref[...] = ex
        ad_ref[...] = ad


def _routing(cp):
    grid = (S // RBLK,)
    order, rank, lo, ex, ad = pl.pallas_call(
        _routing_body,
        grid=grid,
        in_specs=[
            pl.BlockSpec((RBLK, 1), lambda g: (g, 0)),
            pl.BlockSpec((1, S), lambda g: (0, 0)),
        ],
        out_specs=[
            pl.BlockSpec((1, S), lambda g: (0, 0)),
            pl.BlockSpec((RBLK, 1), lambda g: (g, 0)),
            pl.BlockSpec((1, S), lambda g: (0, 0)),
            pl.BlockSpec((1, S), lambda g: (0, 0)),
            pl.BlockSpec((1, S), lambda g: (0, 0)),
        ],
        out_shape=[
            jax.ShapeDtypeStruct((1, S), jnp.int32),
            jax.ShapeDtypeStruct((S, 1), jnp.int32),
            jax.ShapeDtypeStruct((1, S), jnp.int32),
            jax.ShapeDtypeStruct((1, S), jnp.int32),
            jax.ShapeDtypeStruct((1, S), jnp.int32),
        ],
        scratch_shapes=[
            pltpu.VMEM((1, S), jnp.float32),
            pltpu.VMEM((1, S + 2 * KN), jnp.float32),
            pltpu.VMEM((1, S + 2 * KN), jnp.float32),
        ],
    )(cp.reshape(S, 1), cp.reshape(1, S))
    return order.reshape(S), rank.reshape(S), lo, ex, ad


def _sc_gather(table, idx):
    num_cores, num_subcores = 2, 16
    nw = num_cores * num_subcores
    rows, width = table.shape
    per_w = rows // nw
    mesh = plsc.VectorSubcoreMesh(core_axis_name="c", subcore_axis_name="s",
                                  num_cores=num_cores,
                                  num_subcores=num_subcores)

    @functools.partial(
        pl.kernel,
        out_type=jax.ShapeDtypeStruct((rows, width), table.dtype),
        mesh=mesh,
        scratch_types=[
            pltpu.VMEM((per_w,), jnp.int32),
            pltpu.VMEM((per_w, width), table.dtype),
            pltpu.SemaphoreType.DMA,
        ],
    )
    def gather_k(table_hbm, idx_hbm, out_hbm, idx_v, rows_v, sem):
        wid = lax.axis_index("s") * num_cores + lax.axis_index("c")
        base = wid * per_w
        pltpu.sync_copy(idx_hbm.at[pl.ds(base, per_w)], idx_v)
        pltpu.async_copy(table_hbm.at[idx_v], rows_v, sem).wait()
        pltpu.sync_copy(rows_v, out_hbm.at[pl.ds(base, per_w)])

    return gather_k(table, idx)


def _mm_body(x_ref, w_ref, b_ref, o_ref):
    o_ref[...] = (
        jnp.dot(x_ref[...], w_ref[...], precision=_MM_PREC,
                preferred_element_type=jnp.float32)
        + b_ref[...]
    )


def _matmul_bias(x, w, b):
    k, n = w.shape
    grid = (S // RBLK,)
    return pl.pallas_call(
        _mm_body,
        grid=grid,
        in_specs=[
            pl.BlockSpec((RBLK, k), lambda g: (g, 0)),
            pl.BlockSpec((k, n), lambda g: (0, 0)),
            pl.BlockSpec((1, n), lambda g: (0, 0)),
        ],
        out_specs=pl.BlockSpec((RBLK, n), lambda g: (g, 0)),
        out_shape=jax.ShapeDtypeStruct((S, n), jnp.float32),
    )(x, w, b.reshape(1, n))


def _attn_body(lo_smem, q_ref, k_ref, v_ref, lo_ref, ex_ref, ad_ref, o_ref):
    i = pl.program_id(0)
    i0 = i * QBLK
    start = lo_smem[i0]
    start = jnp.clip((start // 8) * 8 - 8, 0, S - WIN)
    start = pl.multiple_of(start, 8)

    ks = k_ref[pl.ds(start, WIN), :]
    vs = v_ref[pl.ds(start, WIN), :]

    jg = start + lax.broadcasted_iota(jnp.int32, (1, WIN), 1)
    lo = lo_ref[...]
    ex = ex_ref[...]
    ad = ad_ref[...]
    inwin = jnp.logical_and(jg >= lo, jg < lo + KN)
    mask = jnp.logical_or(jnp.logical_and(inwin, jg != ex), jg == ad)

    for h in range(HEADS):
        q = q_ref[:, h * HD:(h + 1) * HD]
        kh = ks[:, h * HD:(h + 1) * HD]
        vh = vs[:, h * HD:(h + 1) * HD]
        scores = lax.dot_general(
            q, kh, (((1,), (1,)), ((), ())),
            precision=_MM_PREC, preferred_element_type=jnp.float32) * SCALE
        scores = jnp.where(mask, scores, -BIG)
        m = jnp.max(scores, axis=1, keepdims=True)
        p = jnp.exp(scores - m)
        p = jnp.where(mask, p, 0.0)
        w = p / jnp.sum(p, axis=1, keepdims=True)
        o_ref[:, h * HD:(h + 1) * HD] = jnp.dot(
            w, vh, precision=_MM_PREC, preferred_element_type=jnp.float32)


def _banded_attention(qkv, lo, ex, ad):
    grid = (S // QBLK,)
    return pl.pallas_call(
        _attn_body,
        grid=grid,
        in_specs=[
            pl.BlockSpec(memory_space=pltpu.SMEM),
            pl.BlockSpec((QBLK, DIM), lambda i: (i, 0)),
            pl.BlockSpec((S, DIM), lambda i: (0, 1)),
            pl.BlockSpec((S, DIM), lambda i: (0, 2)),
            pl.BlockSpec((QBLK, 1), lambda i: (i, 0)),
            pl.BlockSpec((QBLK, 1), lambda i: (i, 0)),
            pl.BlockSpec((QBLK, 1), lambda i: (i, 0)),
        ],
        out_specs=pl.BlockSpec((QBLK, DIM), lambda i: (i, 0)),
        out_shape=jax.ShapeDtypeStruct((S, DIM), jnp.float32),
    )(lo.reshape(S), qkv, qkv, qkv,
      lo.reshape(S, 1), ex.reshape(S, 1), ad.reshape(S, 1))


def kernel(x, cantor_positions, W_qkv, b_qkv, W_out, b_out):
    x2 = x.reshape(S, DIM)

    order, rank, lo, ex, ad = _routing(cantor_positions)

    x_sorted = _sc_gather(x2, order)

    qkv_sorted = _matmul_bias(x_sorted, W_qkv.T, b_qkv)

    attn_sorted = _banded_attention(qkv_sorted, lo, ex, ad)

    out_sorted = _matmul_bias(attn_sorted, W_out.T, b_out)

    out = _sc_gather(out_sorted, rank)
    return out.reshape(1, S, DIM)


# speedup vs baseline: 27.7718x; 1.8085x over previous
import functools
import math

import jax
import jax.numpy as jnp
from jax import lax
from jax.experimental import pallas as pl
from jax.experimental.pallas import tpu as pltpu
from jax.experimental.pallas import tpu_sc as plsc

S = 2048
DIM = 1024
HEADS, HD = 16, 64
KN = 64
SCALE = 1.0 / math.sqrt(HD)
QBLK = 128
WIN = 272
RBLK = 256
BIG = 1.0e30

_MM_PREC = jax.lax.Precision.DEFAULT


def _routing_body(cp_col_ref, cp_row_ref, order_ref, rank_ref, lo_ref,
                  ex_ref, ad_ref, s_acc, pad_ref, ord_pad_ref):
    g = pl.program_id(0)

    @pl.when(g == 0)
    def _init():
        s_acc[...] = jnp.zeros_like(s_acc)
        order_ref[...] = jnp.zeros_like(order_ref)

    ci = cp_col_ref[...]
    cr = cp_row_ref[...]
    jidx = lax.broadcasted_iota(jnp.int32, (1, S), 1)
    iglob = g * RBLK + lax.broadcasted_iota(jnp.int32, (RBLK, 1), 0)

    lt = (cr < ci).astype(jnp.int32)
    eq = jnp.logical_and(cr == ci, jidx < iglob).astype(jnp.int32)
    rank = jnp.sum(lt + eq, axis=1, keepdims=True)
    rank_ref[...] = rank

    onehot = jidx == rank
    s_acc[...] += jnp.sum(jnp.where(onehot, ci, 0.0), axis=0, keepdims=True)
    order_ref[...] += jnp.sum(jnp.where(onehot, iglob, 0), axis=0,
                              keepdims=True)

    @pl.when(g == pl.num_programs(0) - 1)
    def _window_starts():
        s = s_acc[...]
        pad_ref[:, 0:KN] = jnp.full((1, KN), -BIG, jnp.float32)
        pad_ref[:, KN:KN + S] = s
        pad_ref[:, KN + S:] = jnp.full((1, KN), BIG, jnp.float32)
        ord_pad_ref[:, 0:KN] = jnp.full((1, KN), 1e9, jnp.float32)
        ord_pad_ref[:, KN:KN + S] = order_ref[...].astype(jnp.float32)
        ord_pad_ref[:, KN + S:] = jnp.full((1, KN), 1e9, jnp.float32)
        best = jnp.full((1, S), 2.0 * BIG, jnp.float32)
        bt = jnp.zeros((1, S), jnp.int32)
        b_sl = jnp.zeros((1, S), jnp.float32)
        b_sll = jnp.zeros((1, S), jnp.float32)
        b_sr = jnp.zeros((1, S), jnp.float32)
        b_sr2 = jnp.zeros((1, S), jnp.float32)
        b_ol = jnp.zeros((1, S), jnp.float32)
        b_or = jnp.zeros((1, S), jnp.float32)
        for t in range(KN):
            a = pad_ref[:, KN - t:KN - t + S]
            al = pad_ref[:, KN - t - 1:KN - t - 1 + S]
            b = pad_ref[:, KN + KN - 1 - t:KN + KN - 1 - t + S]
            b2 = pad_ref[:, KN + KN - 2 - t:KN + KN - 2 - t + S]
            cost = jnp.maximum(s - a, b - s)
            take = cost < best
            best = jnp.where(take, cost, best)
            bt = jnp.where(take, t, bt)
            b_sl = jnp.where(take, a, b_sl)
            b_sll = jnp.where(take, al, b_sll)
            b_sr = jnp.where(take, b, b_sr)
            b_sr2 = jnp.where(take, b2, b_sr2)
            b_ol = jnp.where(
                take, ord_pad_ref[:, KN - t - 1:KN - t - 1 + S], b_ol)
            b_or = jnp.where(
                take, ord_pad_ref[:, KN + KN - 1 - t:KN + KN - 1 - t + S],
                b_or)
        iidx = lax.broadcasted_iota(jnp.int32, (1, S), 1)
        lo = jnp.clip(iidx - bt, 0, S - KN)
        b_dl = s - b_sll
        b_dr = b_sr - s
        case2 = jnp.logical_and(b_sll == b_sl, (s - b_sl) == best)
        case1 = jnp.logical_and(
            jnp.logical_not(case2),
            jnp.logical_and(b_dl == b_dr, b_ol < b_or))
        case3 = jnp.logical_and(
            jnp.logical_and(b_sr2 == b_sr, b_dr == best),
            jnp.logical_and(b_dl < b_dr,
                            jnp.logical_not(jnp.logical_or(case1, case2))))
        sw_r = jnp.logical_or(case1, case3)
        ex = jnp.where(case2, lo, jnp.where(sw_r, lo + KN - 1, -1))
        ad = jnp.where(jnp.logical_or(sw_r, case2), lo - 1, -1)
        lo_ref[...] = lo
        ex_ref[...] = ex
        ad_ref[...] = ad


def _routing(cp):
    grid = (S // RBLK,)
    order, rank, lo, ex, ad = pl.pallas_call(
        _routing_body,
        grid=grid,
        in_specs=[
            pl.BlockSpec((RBLK, 1), lambda g: (g, 0)),
            pl.BlockSpec((1, S), lambda g: (0, 0)),
        ],
        out_specs=[
            pl.BlockSpec((1, S), lambda g: (0, 0)),
            pl.BlockSpec((RBLK, 1), lambda g: (g, 0)),
            pl.BlockSpec((1, S), lambda g: (0, 0)),
            pl.BlockSpec((1, S), lambda g: (0, 0)),
            pl.BlockSpec((1, S), lambda g: (0, 0)),
        ],
        out_shape=[
            jax.ShapeDtypeStruct((1, S), jnp.int32),
            jax.ShapeDtypeStruct((S, 1), jnp.int32),
            jax.ShapeDtypeStruct((1, S), jnp.int32),
            jax.ShapeDtypeStruct((1, S), jnp.int32),
            jax.ShapeDtypeStruct((1, S), jnp.int32),
        ],
        scratch_shapes=[
            pltpu.VMEM((1, S), jnp.float32),
            pltpu.VMEM((1, S + 2 * KN), jnp.float32),
            pltpu.VMEM((1, S + 2 * KN), jnp.float32),
        ],
    )(cp.reshape(S, 1), cp.reshape(1, S))
    return order.reshape(S), rank.reshape(S), lo, ex, ad


def _sc_gather(table, idx):
    num_cores, num_subcores = 2, 16
    nw = num_cores * num_subcores
    rows, width = table.shape
    per_w = rows // nw
    mesh = plsc.VectorSubcoreMesh(core_axis_name="c", subcore_axis_name="s",
                                  num_cores=num_cores,
                                  num_subcores=num_subcores)

    @functools.partial(
        pl.kernel,
        out_type=jax.ShapeDtypeStruct((rows, width), table.dtype),
        mesh=mesh,
        scratch_types=[
            pltpu.VMEM((per_w,), jnp.int32),
            pltpu.VMEM((per_w, width), table.dtype),
            pltpu.SemaphoreType.DMA,
        ],
    )
    def gather_k(table_hbm, idx_hbm, out_hbm, idx_v, rows_v, sem):
        wid = lax.axis_index("s") * num_cores + lax.axis_index("c")
        base = wid * per_w
        pltpu.sync_copy(idx_hbm.at[pl.ds(base, per_w)], idx_v)
        pltpu.async_copy(table_hbm.at[idx_v], rows_v, sem).wait()
        pltpu.sync_copy(rows_v, out_hbm.at[pl.ds(base, per_w)])

    return gather_k(table, idx)


def _mm_body(x_ref, w_ref, b_ref, o_ref):
    o_ref[...] = (
        jnp.dot(x_ref[...], w_ref[...], precision=_MM_PREC,
                preferred_element_type=jnp.float32)
        + b_ref[...]
    )


def _matmul_bias(x, w, b):
    k, n = w.shape
    grid = (S // RBLK,)
    return pl.pallas_call(
        _mm_body,
        grid=grid,
        in_specs=[
            pl.BlockSpec((RBLK, k), lambda g: (g, 0)),
            pl.BlockSpec((k, n), lambda g: (0, 0)),
            pl.BlockSpec((1, n), lambda g: (0, 0)),
        ],
        out_specs=pl.BlockSpec((RBLK, n), lambda g: (g, 0)),
        out_shape=jax.ShapeDtypeStruct((S, n), jnp.float32),
    )(x, w, b.reshape(1, n))


def _attn_body(lo_smem, q_ref, k_ref, v_ref, lo_ref, ex_ref, ad_ref, o_ref):
    i = pl.program_id(0)
    i0 = i * QBLK
    start = lo_smem[i0]
    start = jnp.clip((start // 8) * 8 - 8, 0, S - WIN)
    start = pl.multiple_of(start, 8)

    ks = k_ref[pl.ds(start, WIN), :]
    vs = v_ref[pl.ds(start, WIN), :]

    jg = start + lax.broadcasted_iota(jnp.int32, (1, WIN), 1)
    lo = lo_ref[...]
    ex = ex_ref[...]
    ad = ad_ref[...]
    inwin = jnp.logical_and(jg >= lo, jg < lo + KN)
    mask = jnp.logical_or(jnp.logical_and(inwin, jg != ex), jg == ad)

    for h in range(HEADS):
        q = q_ref[:, h * HD:(h + 1) * HD]
        kh = ks[:, h * HD:(h + 1) * HD]
        vh = vs[:, h * HD:(h + 1) * HD]
        scores = lax.dot_general(
            q, kh, (((1,), (1,)), ((), ())),
            precision=_MM_PREC, preferred_element_type=jnp.float32) * SCALE
        scores = jnp.where(mask, scores, -BIG)
        m = jnp.max(scores, axis=1, keepdims=True)
        p = jnp.exp(scores - m)
        p = jnp.where(mask, p, 0.0)
        w = p / jnp.sum(p, axis=1, keepdims=True)
        o_ref[:, h * HD:(h + 1) * HD] = jnp.dot(
            w, vh, precision=_MM_PREC, preferred_element_type=jnp.float32)


def _banded_attention(qkv, lo, ex, ad):
    grid = (S // QBLK,)
    return pl.pallas_call(
        _attn_body,
        grid=grid,
        in_specs=[
            pl.BlockSpec(memory_space=pltpu.SMEM),
            pl.BlockSpec((QBLK, DIM), lambda i: (i, 0)),
            pl.BlockSpec((S, DIM), lambda i: (0, 1)),
            pl.BlockSpec((S, DIM), lambda i: (0, 2)),
            pl.BlockSpec((QBLK, 1), lambda i: (i, 0)),
            pl.BlockSpec((QBLK, 1), lambda i: (i, 0)),
            pl.BlockSpec((QBLK, 1), lambda i: (i, 0)),
        ],
        out_specs=pl.BlockSpec((QBLK, DIM), lambda i: (i, 0)),
        out_shape=jax.ShapeDtypeStruct((S, DIM), jnp.float32),
    )(lo.reshape(S), qkv, qkv, qkv,
      lo.reshape(S, 1), ex.reshape(S, 1), ad.reshape(S, 1))


def kernel(x, cantor_positions, W_qkv, b_qkv, W_out, b_out):
    x2 = x.reshape(S, DIM)

    order, rank, lo, ex, ad = _routing(cantor_positions)

    x_sorted = _sc_gather(x2, order)

    qkv_sorted = _matmul_bias(x_sorted, W_qkv.T, b_qkv)

    attn_sorted = _banded_attention(qkv_sorted, lo, ex, ad)

    out_sorted = _matmul_bias(attn_sorted, W_out.T, b_out)

    out = _sc_gather(out_sorted, rank)
    return out.reshape(1, S, DIM)


# speedup vs baseline: 36.7570x; 1.3235x over previous
import functools
import math

import jax
import jax.numpy as jnp
from jax import lax
from jax.experimental import pallas as pl
from jax.experimental.pallas import tpu as pltpu
from jax.experimental.pallas import tpu_sc as plsc

S = 2048
DIM = 1024
HEADS, HD = 16, 64
KN = 64
SCALE = 1.0 / math.sqrt(HD)
QBLK = 256
WIN = QBLK + 144
RBLK = 256
BIG = 1.0e30

_MM_PREC = jax.lax.Precision.DEFAULT


def _routing_body(cp_col_ref, cp_row_ref, order_ref, rank_ref, lo_ref,
                  ex_ref, ad_ref, s_acc, pad_ref, ord_pad_ref):
    g = pl.program_id(0)

    @pl.when(g == 0)
    def _init():
        s_acc[...] = jnp.zeros_like(s_acc)
        order_ref[...] = jnp.zeros_like(order_ref)

    ci = cp_col_ref[...]
    cr = cp_row_ref[...]
    jidx = lax.broadcasted_iota(jnp.int32, (1, S), 1)
    iglob = g * RBLK + lax.broadcasted_iota(jnp.int32, (RBLK, 1), 0)

    lt = (cr < ci).astype(jnp.int32)
    eq = jnp.logical_and(cr == ci, jidx < iglob).astype(jnp.int32)
    rank = jnp.sum(lt + eq, axis=1, keepdims=True)
    rank_ref[...] = rank

    onehot = jidx == rank
    s_acc[...] += jnp.sum(jnp.where(onehot, ci, 0.0), axis=0, keepdims=True)
    order_ref[...] += jnp.sum(jnp.where(onehot, iglob, 0), axis=0,
                              keepdims=True)

    @pl.when(g == pl.num_programs(0) - 1)
    def _window_starts():
        s = s_acc[...]
        pad_ref[:, 0:KN] = jnp.full((1, KN), -BIG, jnp.float32)
        pad_ref[:, KN:KN + S] = s
        pad_ref[:, KN + S:] = jnp.full((1, KN), BIG, jnp.float32)
        ord_pad_ref[:, 0:KN] = jnp.full((1, KN), 1e9, jnp.float32)
        ord_pad_ref[:, KN:KN + S] = order_ref[...].astype(jnp.float32)
        ord_pad_ref[:, KN + S:] = jnp.full((1, KN), 1e9, jnp.float32)
        best = jnp.full((1, S), 2.0 * BIG, jnp.float32)
        bt = jnp.zeros((1, S), jnp.int32)
        b_sl = jnp.zeros((1, S), jnp.float32)
        b_sll = jnp.zeros((1, S), jnp.float32)
        b_sr = jnp.zeros((1, S), jnp.float32)
        b_sr2 = jnp.zeros((1, S), jnp.float32)
        b_ol = jnp.zeros((1, S), jnp.float32)
        b_or = jnp.zeros((1, S), jnp.float32)
        for t in range(KN):
            a = pad_ref[:, KN - t:KN - t + S]
            al = pad_ref[:, KN - t - 1:KN - t - 1 + S]
            b = pad_ref[:, KN + KN - 1 - t:KN + KN - 1 - t + S]
            b2 = pad_ref[:, KN + KN - 2 - t:KN + KN - 2 - t + S]
            cost = jnp.maximum(s - a, b - s)
            take = cost < best
            best = jnp.where(take, cost, best)
            bt = jnp.where(take, t, bt)
            b_sl = jnp.where(take, a, b_sl)
            b_sll = jnp.where(take, al, b_sll)
            b_sr = jnp.where(take, b, b_sr)
            b_sr2 = jnp.where(take, b2, b_sr2)
            b_ol = jnp.where(
                take, ord_pad_ref[:, KN - t - 1:KN - t - 1 + S], b_ol)
            b_or = jnp.where(
                take, ord_pad_ref[:, KN + KN - 1 - t:KN + KN - 1 - t + S],
                b_or)
        iidx = lax.broadcasted_iota(jnp.int32, (1, S), 1)
        lo = jnp.clip(iidx - bt, 0, S - KN)
        b_dl = s - b_sll
        b_dr = b_sr - s
        case2 = jnp.logical_and(b_sll == b_sl, (s - b_sl) == best)
        case1 = jnp.logical_and(
            jnp.logical_not(case2),
            jnp.logical_and(b_dl == b_dr, b_ol < b_or))
        case3 = jnp.logical_and(
            jnp.logical_and(b_sr2 == b_sr, b_dr == best),
            jnp.logical_and(b_dl < b_dr,
                            jnp.logical_not(jnp.logical_or(case1, case2))))
        sw_r = jnp.logical_or(case1, case3)
        ex = jnp.where(case2, lo, jnp.where(sw_r, lo + KN - 1, -1))
        ad = jnp.where(jnp.logical_or(sw_r, case2), lo - 1, -1)
        lo_ref[...] = lo
        ex_ref[...] = ex
        ad_ref[...] = ad


def _routing(cp):
    grid = (S // RBLK,)
    order, rank, lo, ex, ad = pl.pallas_call(
        _routing_body,
        grid=grid,
        in_specs=[
            pl.BlockSpec((RBLK, 1), lambda g: (g, 0)),
            pl.BlockSpec((1, S), lambda g: (0, 0)),
        ],
        out_specs=[
            pl.BlockSpec((1, S), lambda g: (0, 0)),
            pl.BlockSpec((RBLK, 1), lambda g: (g, 0)),
            pl.BlockSpec((1, S), lambda g: (0, 0)),
            pl.BlockSpec((1, S), lambda g: (0, 0)),
            pl.BlockSpec((1, S), lambda g: (0, 0)),
        ],
        out_shape=[
            jax.ShapeDtypeStruct((1, S), jnp.int32),
            jax.ShapeDtypeStruct((S, 1), jnp.int32),
            jax.ShapeDtypeStruct((1, S), jnp.int32),
            jax.ShapeDtypeStruct((1, S), jnp.int32),
            jax.ShapeDtypeStruct((1, S), jnp.int32),
        ],
        scratch_shapes=[
            pltpu.VMEM((1, S), jnp.float32),
            pltpu.VMEM((1, S + 2 * KN), jnp.float32),
            pltpu.VMEM((1, S + 2 * KN), jnp.float32),
        ],
    )(cp.reshape(S, 1), cp.reshape(1, S))
    return order.reshape(S), rank.reshape(S), lo, ex, ad


def _sc_gather(table, idx):
    num_cores, num_subcores = 2, 16
    nw = num_cores * num_subcores
    rows, width = table.shape
    per_w = rows // nw
    mesh = plsc.VectorSubcoreMesh(core_axis_name="c", subcore_axis_name="s",
                                  num_cores=num_cores,
                                  num_subcores=num_subcores)

    @functools.partial(
        pl.kernel,
        out_type=jax.ShapeDtypeStruct((rows, width), table.dtype),
        mesh=mesh,
        scratch_types=[
            pltpu.VMEM((per_w,), jnp.int32),
            pltpu.VMEM((per_w, width), table.dtype),
            pltpu.SemaphoreType.DMA,
        ],
    )
    def gather_k(table_hbm, idx_hbm, out_hbm, idx_v, rows_v, sem):
        wid = lax.axis_index("s") * num_cores + lax.axis_index("c")
        base = wid * per_w
        pltpu.sync_copy(idx_hbm.at[pl.ds(base, per_w)], idx_v)
        pltpu.async_copy(table_hbm.at[idx_v], rows_v, sem).wait()
        pltpu.sync_copy(rows_v, out_hbm.at[pl.ds(base, per_w)])

    return gather_k(table, idx)


def _mm_body(x_ref, w_ref, b_ref, o_ref):
    o_ref[...] = lax.dot_general(
        x_ref[...], w_ref[...], (((1,), (1,)), ((), ())),
        precision=_MM_PREC, preferred_element_type=jnp.float32) + b_ref[...]


def _matmul_bias(x, w, b):
    n, k = w.shape
    grid = (S // RBLK,)
    return pl.pallas_call(
        _mm_body,
        grid=grid,
        in_specs=[
            pl.BlockSpec((RBLK, k), lambda g: (g, 0)),
            pl.BlockSpec((n, k), lambda g: (0, 0)),
            pl.BlockSpec((1, n), lambda g: (0, 0)),
        ],
        out_specs=pl.BlockSpec((RBLK, n), lambda g: (g, 0)),
        out_shape=jax.ShapeDtypeStruct((S, n), jnp.float32),
    )(x, w, b.reshape(1, n))


def _attn_body(lo_smem, q_ref, k_ref, v_ref, lo_ref, ex_ref, ad_ref,
               wo_ref, bo_ref, o_ref, acc_ref):
    i = pl.program_id(0)
    i0 = i * QBLK
    start = lo_smem[i0]
    start = jnp.clip((start // 8) * 8 - 8, 0, S - WIN)
    start = pl.multiple_of(start, 8)

    ks = k_ref[pl.ds(start, WIN), :]
    vs = v_ref[pl.ds(start, WIN), :]

    jg = start + lax.broadcasted_iota(jnp.int32, (1, WIN), 1)
    lo = lo_ref[...]
    ex = ex_ref[...]
    ad = ad_ref[...]
    inwin = jnp.logical_and(jg >= lo, jg < lo + KN)
    mask = jnp.logical_or(jnp.logical_and(inwin, jg != ex), jg == ad)

    for h in range(HEADS):
        q = q_ref[:, h * HD:(h + 1) * HD]
        kh = ks[:, h * HD:(h + 1) * HD]
        vh = vs[:, h * HD:(h + 1) * HD]
        scores = lax.dot_general(
            q, kh, (((1,), (1,)), ((), ())),
            precision=_MM_PREC, preferred_element_type=jnp.float32) * SCALE
        scores = jnp.where(mask, scores, -BIG)
        m = jnp.max(scores, axis=1, keepdims=True)
        p = jnp.exp(scores - m)
        p = jnp.where(mask, p, 0.0)
        w = p / jnp.sum(p, axis=1, keepdims=True)
        acc_ref[:, h * HD:(h + 1) * HD] = jnp.dot(
            w, vh, precision=_MM_PREC, preferred_element_type=jnp.float32)

    o_ref[...] = lax.dot_general(
        acc_ref[...], wo_ref[...], (((1,), (1,)), ((), ())),
        precision=_MM_PREC, preferred_element_type=jnp.float32) + bo_ref[...]


def _banded_attention(qkv, lo, ex, ad, w_out, b_out):
    grid = (S // QBLK,)
    return pl.pallas_call(
        _attn_body,
        grid=grid,
        in_specs=[
            pl.BlockSpec(memory_space=pltpu.SMEM),
            pl.BlockSpec((QBLK, DIM), lambda i: (i, 0)),
            pl.BlockSpec((S, DIM), lambda i: (0, 1)),
            pl.BlockSpec((S, DIM), lambda i: (0, 2)),
            pl.BlockSpec((QBLK, 1), lambda i: (i, 0)),
            pl.BlockSpec((QBLK, 1), lambda i: (i, 0)),
            pl.BlockSpec((QBLK, 1), lambda i: (i, 0)),
            pl.BlockSpec((DIM, DIM), lambda i: (0, 0)),
            pl.BlockSpec((1, DIM), lambda i: (0, 0)),
        ],
        out_specs=pl.BlockSpec((QBLK, DIM), lambda i: (i, 0)),
        out_shape=jax.ShapeDtypeStruct((S, DIM), jnp.float32),
        scratch_shapes=[pltpu.VMEM((QBLK, DIM), jnp.float32)],
    )(lo.reshape(S), qkv, qkv, qkv,
      lo.reshape(S, 1), ex.reshape(S, 1), ad.reshape(S, 1),
      w_out, b_out.reshape(1, DIM))


def kernel(x, cantor_positions, W_qkv, b_qkv, W_out, b_out):
    x2 = x.reshape(S, DIM)

    order, rank, lo, ex, ad = _routing(cantor_positions)

    x_sorted = _sc_gather(x2, order)

    qkv_sorted = _matmul_bias(x_sorted, W_qkv, b_qkv)

    out_sorted = _banded_attention(qkv_sorted, lo, ex, ad, W_out, b_out)

    out = _sc_gather(out_sorted, rank)
    return out.reshape(1, S, DIM)


# speedup vs baseline: 42.0450x; 1.1439x over previous
import functools
import math

import jax
import jax.numpy as jnp
from jax import lax
from jax.experimental import pallas as pl
from jax.experimental.pallas import tpu as pltpu
from jax.experimental.pallas import tpu_sc as plsc

S = 2048
DIM = 1024
HEADS, HD = 16, 64
KN = 64
SCALE = 1.0 / math.sqrt(HD)
QBLK = 256
WIN = QBLK + 144
RBLK = 256
BIG = 1.0e30

_MM_PREC = jax.lax.Precision.DEFAULT


def _routing_body(cp_col_ref, cp_row_ref, order_ref, rank_ref, lo_ref,
                  ex_ref, ad_ref, s_acc, pad_ref, ord_pad_ref):
    g = pl.program_id(0)

    @pl.when(g == 0)
    def _init():
        s_acc[...] = jnp.zeros_like(s_acc)
        order_ref[...] = jnp.zeros_like(order_ref)

    ci = cp_col_ref[...]
    cr = cp_row_ref[...]
    jidx = lax.broadcasted_iota(jnp.int32, (1, S), 1)
    iglob = g * RBLK + lax.broadcasted_iota(jnp.int32, (RBLK, 1), 0)

    lt = jnp.logical_or(
        cr < ci, jnp.logical_and(cr == ci, jidx < iglob)).astype(jnp.float32)
    ones = jnp.ones((S, 1), jnp.float32)
    rankf = lax.dot_general(
        lt, ones, (((1,), (0,)), ((), ())),
        precision=_MM_PREC, preferred_element_type=jnp.float32)
    rank = rankf.astype(jnp.int32)
    rank_ref[...] = rank

    onehot = (jidx == rank).astype(jnp.float32)
    vals = jnp.concatenate([ci, iglob.astype(jnp.float32)], axis=1)
    both = lax.dot_general(
        vals, onehot, (((0,), (0,)), ((), ())),
        precision=jax.lax.Precision.HIGHEST, preferred_element_type=jnp.float32)
    s_acc[...] += both[0:1, :]
    order_ref[...] += both[1:2, :].astype(jnp.int32)

    @pl.when(g == pl.num_programs(0) - 1)
    def _window_starts():
        s = s_acc[...]
        pad_ref[:, 0:KN] = jnp.full((1, KN), -BIG, jnp.float32)
        pad_ref[:, KN:KN + S] = s
        pad_ref[:, KN + S:] = jnp.full((1, KN), BIG, jnp.float32)
        ord_pad_ref[:, 0:KN] = jnp.full((1, KN), 1e9, jnp.float32)
        ord_pad_ref[:, KN:KN + S] = order_ref[...].astype(jnp.float32)
        ord_pad_ref[:, KN + S:] = jnp.full((1, KN), 1e9, jnp.float32)
        best = jnp.full((1, S), 2.0 * BIG, jnp.float32)
        bt = jnp.zeros((1, S), jnp.int32)
        b_sl = jnp.zeros((1, S), jnp.float32)
        b_sll = jnp.zeros((1, S), jnp.float32)
        b_sr = jnp.zeros((1, S), jnp.float32)
        b_sr2 = jnp.zeros((1, S), jnp.float32)
        b_ol = jnp.zeros((1, S), jnp.float32)
        b_or = jnp.zeros((1, S), jnp.float32)
        for t in range(KN):
            a = pad_ref[:, KN - t:KN - t + S]
            al = pad_ref[:, KN - t - 1:KN - t - 1 + S]
            b = pad_ref[:, KN + KN - 1 - t:KN + KN - 1 - t + S]
            b2 = pad_ref[:, KN + KN - 2 - t:KN + KN - 2 - t + S]
            cost = jnp.maximum(s - a, b - s)
            take = cost < best
            best = jnp.where(take, cost, best)
            bt = jnp.where(take, t, bt)
            b_sl = jnp.where(take, a, b_sl)
            b_sll = jnp.where(take, al, b_sll)
            b_sr = jnp.where(take, b, b_sr)
            b_sr2 = jnp.where(take, b2, b_sr2)
            b_ol = jnp.where(
                take, ord_pad_ref[:, KN - t - 1:KN - t - 1 + S], b_ol)
            b_or = jnp.where(
                take, ord_pad_ref[:, KN + KN - 1 - t:KN + KN - 1 - t + S],
                b_or)
        iidx = lax.broadcasted_iota(jnp.int32, (1, S), 1)
        lo = jnp.clip(iidx - bt, 0, S - KN)
        b_dl = s - b_sll
        b_dr = b_sr - s
        case2 = jnp.logical_and(b_sll == b_sl, (s - b_sl) == best)
        case1 = jnp.logical_and(
            jnp.logical_not(case2),
            jnp.logical_and(b_dl == b_dr, b_ol < b_or))
        case3 = jnp.logical_and(
            jnp.logical_and(b_sr2 == b_sr, b_dr == best),
            jnp.logical_and(b_dl < b_dr,
                            jnp.logical_not(jnp.logical_or(case1, case2))))
        sw_r = jnp.logical_or(case1, case3)
        ex = jnp.where(case2, lo, jnp.where(sw_r, lo + KN - 1, -1))
        ad = jnp.where(jnp.logical_or(sw_r, case2), lo - 1, -1)
        lo_ref[...] = lo
        ex_ref[...] = ex
        ad_ref[...] = ad


def _routing(cp):
    grid = (S // RBLK,)
    order, rank, lo, ex, ad = pl.pallas_call(
        _routing_body,
        grid=grid,
        in_specs=[
            pl.BlockSpec((RBLK, 1), lambda g: (g, 0)),
            pl.BlockSpec((1, S), lambda g: (0, 0)),
        ],
        out_specs=[
            pl.BlockSpec((1, S), lambda g: (0, 0)),
            pl.BlockSpec((RBLK, 1), lambda g: (g, 0)),
            pl.BlockSpec((1, S), lambda g: (0, 0)),
            pl.BlockSpec((1, S), lambda g: (0, 0)),
            pl.BlockSpec((1, S), lambda g: (0, 0)),
        ],
        out_shape=[
            jax.ShapeDtypeStruct((1, S), jnp.int32),
            jax.ShapeDtypeStruct((S, 1), jnp.int32),
            jax.ShapeDtypeStruct((1, S), jnp.int32),
            jax.ShapeDtypeStruct((1, S), jnp.int32),
            jax.ShapeDtypeStruct((1, S), jnp.int32),
        ],
        scratch_shapes=[
            pltpu.VMEM((1, S), jnp.float32),
            pltpu.VMEM((1, S + 2 * KN), jnp.float32),
            pltpu.VMEM((1, S + 2 * KN), jnp.float32),
        ],
    )(cp.reshape(S, 1), cp.reshape(1, S))
    return order.reshape(S), rank.reshape(S), lo, ex, ad


def _sc_gather(table, idx):
    num_cores, num_subcores = 2, 16
    nw = num_cores * num_subcores
    rows, width = table.shape
    per_w = rows // nw
    mesh = plsc.VectorSubcoreMesh(core_axis_name="c", subcore_axis_name="s",
                                  num_cores=num_cores,
                                  num_subcores=num_subcores)

    @functools.partial(
        pl.kernel,
        out_type=jax.ShapeDtypeStruct((rows, width), table.dtype),
        mesh=mesh,
        scratch_types=[
            pltpu.VMEM((per_w,), jnp.int32),
            pltpu.VMEM((per_w, width), table.dtype),
            pltpu.SemaphoreType.DMA,
        ],
    )
    def gather_k(table_hbm, idx_hbm, out_hbm, idx_v, rows_v, sem):
        wid = lax.axis_index("s") * num_cores + lax.axis_index("c")
        base = wid * per_w
        pltpu.sync_copy(idx_hbm.at[pl.ds(base, per_w)], idx_v)
        pltpu.async_copy(table_hbm.at[idx_v], rows_v, sem).wait()
        pltpu.sync_copy(rows_v, out_hbm.at[pl.ds(base, per_w)])

    return gather_k(table, idx)


def _mm_body(x_ref, w_ref, b_ref, o_ref):
    o_ref[...] = lax.dot_general(
        x_ref[...], w_ref[...], (((1,), (1,)), ((), ())),
        precision=_MM_PREC, preferred_element_type=jnp.float32) + b_ref[...]


def _matmul_bias(x, w, b):
    n, k = w.shape
    grid = (S // RBLK,)
    return pl.pallas_call(
        _mm_body,
        grid=grid,
        in_specs=[
            pl.BlockSpec((RBLK, k), lambda g: (g, 0)),
            pl.BlockSpec((n, k), lambda g: (0, 0)),
            pl.BlockSpec((1, n), lambda g: (0, 0)),
        ],
        out_specs=pl.BlockSpec((RBLK, n), lambda g: (g, 0)),
        out_shape=jax.ShapeDtypeStruct((S, n), jnp.float32),
    )(x, w, b.reshape(1, n))


def _attn_body(lo_smem, q_ref, k_ref, v_ref, lo_ref, ex_ref, ad_ref,
               wo_ref, bo_ref, o_ref, acc_ref):
    i = pl.program_id(0)
    i0 = i * QBLK
    start = lo_smem[i0]
    start = jnp.clip((start // 8) * 8 - 8, 0, S - WIN)
    start = pl.multiple_of(start, 8)

    ks = k_ref[pl.ds(start, WIN), :]
    vs = v_ref[pl.ds(start, WIN), :]

    jg = start + lax.broadcasted_iota(jnp.int32, (1, WIN), 1)
    lo = lo_ref[...]
    ex = ex_ref[...]
    ad = ad_ref[...]
    inwin = jnp.logical_and(jg >= lo, jg < lo + KN)
    mask = jnp.logical_or(jnp.logical_and(inwin, jg != ex), jg == ad)
    maskf = mask.astype(jnp.float32)

    qs = q_ref[...] * SCALE
    for h in range(HEADS):
        q = qs[:, h * HD:(h + 1) * HD]
        kh = ks[:, h * HD:(h + 1) * HD]
        vh = vs[:, h * HD:(h + 1) * HD]
        scores = lax.dot_general(
            q, kh, (((1,), (1,)), ((), ())),
            precision=_MM_PREC, preferred_element_type=jnp.float32)
        p = jnp.exp(scores) * maskf
        pv = lax.dot_general(
            p, vh, (((1,), (0,)), ((), ())),
            precision=_MM_PREC, preferred_element_type=jnp.float32)
        denom = jnp.sum(p, axis=1, keepdims=True)
        acc_ref[:, h * HD:(h + 1) * HD] = pv / denom

    o_ref[...] = lax.dot_general(
        acc_ref[...], wo_ref[...], (((1,), (1,)), ((), ())),
        precision=_MM_PREC, preferred_element_type=jnp.float32) + bo_ref[...]


def _banded_attention(qkv, lo, ex, ad, w_out, b_out):
    grid = (S // QBLK,)
    return pl.pallas_call(
        _attn_body,
        grid=grid,
        in_specs=[
            pl.BlockSpec(memory_space=pltpu.SMEM),
            pl.BlockSpec((QBLK, DIM), lambda i: (i, 0)),
            pl.BlockSpec((S, DIM), lambda i: (0, 1)),
            pl.BlockSpec((S, DIM), lambda i: (0, 2)),
            pl.BlockSpec((QBLK, 1), lambda i: (i, 0)),
            pl.BlockSpec((QBLK, 1), lambda i: (i, 0)),
            pl.BlockSpec((QBLK, 1), lambda i: (i, 0)),
            pl.BlockSpec((DIM, DIM), lambda i: (0, 0)),
            pl.BlockSpec((1, DIM), lambda i: (0, 0)),
        ],
        out_specs=pl.BlockSpec((QBLK, DIM), lambda i: (i, 0)),
        out_shape=jax.ShapeDtypeStruct((S, DIM), jnp.float32),
        scratch_shapes=[pltpu.VMEM((QBLK, DIM), jnp.float32)],
    )(lo.reshape(S), qkv, qkv, qkv,
      lo.reshape(S, 1), ex.reshape(S, 1), ad.reshape(S, 1),
      w_out, b_out.reshape(1, DIM))


def kernel(x, cantor_positions, W_qkv, b_qkv, W_out, b_out):
    x2 = x.reshape(S, DIM)

    order, rank, lo, ex, ad = _routing(cantor_positions)

    x_sorted = _sc_gather(x2, order)

    qkv_sorted = _matmul_bias(x_sorted, W_qkv, b_qkv)

    out_sorted = _banded_attention(qkv_sorted, lo, ex, ad, W_out, b_out)

    out = _sc_gather(out_sorted, rank)
    return out.reshape(1, S, DIM)
